# baseline (speedup 1.0000x reference)
import functools

import jax
import jax.numpy as jnp
from jax import lax
from jax.experimental import pallas as pl
from jax.experimental.pallas import tpu as pltpu
from jax.experimental.pallas import tpu_sc as plsc

N = 10000
E = 320000
D = 128
H1 = 1546
NC = 2
NS = 16
NW = NC * NS
EPW = E // NW
C = 40
NCHUNK = EPW // C
CP = 48
C1 = 80
NCH1 = EPW // C1
RPT = 624
REM = N - RPT * NS

_SC_MESH = plsc.VectorSubcoreMesh(core_axis_name="c", subcore_axis_name="s")


def _shuffle(v, sh):
    idx = (jnp.arange(16, dtype=jnp.int32) ^ sh)[:, None]
    return lax.gather(
        v, idx,
        dimension_numbers=lax.GatherDimensionNumbers(
            offset_dims=(), collapsed_slice_dims=(0,), start_index_map=(0,)),
        slice_sizes=(1,),
        mode=lax.GatherScatterMode.PROMISE_IN_BOUNDS)


def _lane_sum(v):
    for sh in (1, 2, 4, 8):
        v = v + _shuffle(v, sh)
    return v


def _lane_max(v):
    for sh in (1, 2, 4, 8):
        v = jnp.maximum(v, _shuffle(v, sh))
    return v


_BLK = 1000
_GRID = N // _BLK


def _proj_body(x_ref, w_ref, b_ref, wl_ref, wl2_ref, bl_ref, bl2_ref,
               q_ref, k_ref, v_ref, hi_ref, hj_ref, root_ref, wf_ref, bf_ref):
    y = jnp.dot(x_ref[...], w_ref[...], preferred_element_type=jnp.float32)
    y = y + b_ref[...]
    q_ref[...] = y[:, 0:128]
    k_ref[...] = y[:, 128:256]
    v_ref[...] = y[:, 256:384]
    hi_ref[...] = y[:, 384:512]
    hj_ref[...] = y[:, 512:640]
    root_ref[...] = y[:, 640:768]

    @pl.when(pl.program_id(0) == 0)
    def _fuse():
        wf_ref[...] = jnp.dot(wl_ref[...], wl2_ref[...],
                              preferred_element_type=jnp.float32)
        bf_ref[...] = jnp.dot(bl_ref[...], wl2_ref[...],
                              preferred_element_type=jnp.float32) + bl2_ref[...]


def _proj(x, w_all, b_all, Wl, Wl2, bl, bl2):
    outs = ([jax.ShapeDtypeStruct((N, D), jnp.float32) for _ in range(6)]
            + [jax.ShapeDtypeStruct((D, D), jnp.float32),
               jax.ShapeDtypeStruct((1, D), jnp.float32)])
    return pl.pallas_call(
        _proj_body,
        grid=(_GRID,),
        in_specs=[
            pl.BlockSpec((_BLK, D), lambda i: (i, 0)),
            pl.BlockSpec((D, 768), lambda i: (0, 0)),
            pl.BlockSpec((1, 768), lambda i: (0, 0)),
            pl.BlockSpec((D, H1), lambda i: (0, 0)),
            pl.BlockSpec((H1, D), lambda i: (0, 0)),
            pl.BlockSpec((1, H1), lambda i: (0, 0)),
            pl.BlockSpec((1, D), lambda i: (0, 0)),
        ],
        out_specs=[pl.BlockSpec((_BLK, D), lambda i: (i, 0))] * 6
        + [pl.BlockSpec((D, D), lambda i: (0, 0)),
           pl.BlockSpec((1, D), lambda i: (0, 0))],
        out_shape=outs,
    )(x, w_all, b_all, Wl, Wl2, bl, bl2)


def _maybe_when(pred, fn):
    if isinstance(pred, bool):
        if pred:
            fn()
        return
    pl.when(pred)(fn)


@functools.partial(
    pl.kernel,
    out_type=[
        jax.ShapeDtypeStruct((E,), jnp.float32),
        jax.ShapeDtypeStruct((NW * 16,), jnp.float32),
        jax.ShapeDtypeStruct((NW * 16,), jnp.float32),
    ],
    mesh=_SC_MESH,
    scratch_types=[
        pltpu.VMEM((2, C1), jnp.int32),
        pltpu.VMEM((2, C1), jnp.int32),
        pltpu.VMEM((2, C1, D), jnp.float32),
        pltpu.VMEM((2, C1, D), jnp.float32),
        pltpu.VMEM((2, C1 + 16), jnp.float32),
        pltpu.VMEM((16,), jnp.float32),
        pltpu.SemaphoreType.DMA,
        pltpu.SemaphoreType.DMA,
        pltpu.SemaphoreType.DMA,
        pltpu.SemaphoreType.DMA,
        pltpu.SemaphoreType.DMA,
        pltpu.SemaphoreType.DMA,
    ],
)
def _s1(src_hbm, dst_hbm, q_hbm, k_hbm, att_out, m_out, s_out,
        sidx, didx, qrows, krows, attb, statb,
        sem_i0, sem_i1, sem_g0, sem_g1, sem_w0, sem_w1):
    cid = lax.axis_index("c")
    sid = lax.axis_index("s")
    wid = sid * NC + cid
    base_w = wid * EPW
    sem_i = (sem_i0, sem_i1)
    sem_g = (sem_g0, sem_g1)
    sem_w = (sem_w0, sem_w1)
    lane = jnp.arange(16, dtype=jnp.int32)

    def issue_idx(i, slot):
        base = base_w + i * C1
        pltpu.async_copy(src_hbm.at[pl.ds(base, C1)], sidx.at[slot],
                         sem_i[slot])
        pltpu.async_copy(dst_hbm.at[pl.ds(base, C1)], didx.at[slot],
                         sem_i[slot])

    def wait_idx(slot):
        pltpu.make_async_copy(src_hbm.at[pl.ds(0, C1)], sidx.at[slot],
                              sem_i[slot]).wait()
        pltpu.make_async_copy(dst_hbm.at[pl.ds(0, C1)], didx.at[slot],
                              sem_i[slot]).wait()

    def issue_gathers(slot):
        pltpu.async_copy(q_hbm.at[didx.at[slot]], qrows.at[slot], sem_g[slot])
        pltpu.async_copy(k_hbm.at[sidx.at[slot]], krows.at[slot], sem_g[slot])

    def wait_gathers(slot):
        pltpu.make_async_copy(q_hbm.at[didx.at[slot]], qrows.at[slot],
                              sem_g[slot]).wait()
        pltpu.make_async_copy(k_hbm.at[sidx.at[slot]], krows.at[slot],
                              sem_g[slot]).wait()

    def do_iter(i, b, carry, p_idx2, p_attw):
        m_r, s_r = carry
        nb = 1 - b
        if not isinstance(i, int) or i + 1 < NCH1:
            wait_idx(nb)
            issue_gathers(nb)
        _maybe_when(p_attw, lambda: pltpu.make_async_copy(
            attb.at[b, pl.ds(0, C1)], att_out.at[pl.ds(0, C1)],
            sem_w[b]).wait())
        wait_gathers(b)
        _maybe_when(p_idx2, lambda: issue_idx(i + 2, b))

        def one_edge(j):
            acc = qrows[b, j, 0:16] * krows[b, j, 0:16]
            for r in range(1, 8):
                acc = acc + (qrows[b, j, 16 * r:16 * r + 16]
                             * krows[b, j, 16 * r:16 * r + 16])
            attb[b, pl.ds(j, 16)] = _lane_sum(acc)

        def edge(jj, _):
            one_edge(jj * 2)
            one_edge(jj * 2 + 1)
            return 0

        lax.fori_loop(0, C1 // 2, edge, 0)
        for t in range(C1 // 16):
            vec = attb[b, t * 16:t * 16 + 16]
            m_n = jnp.maximum(m_r, vec)
            s_r = s_r * jnp.exp(m_r - m_n) + jnp.exp(vec - m_n)
            m_r = m_n
        pltpu.async_copy(attb.at[b, pl.ds(0, C1)],
                         att_out.at[pl.ds(base_w + i * C1, C1)], sem_w[b])
        return (m_r, s_r)

    issue_idx(0, 0)
    issue_idx(1, 1)
    wait_idx(0)
    issue_gathers(0)

    m0 = jnp.full((16,), -1e30, dtype=jnp.float32)
    s0 = jnp.zeros((16,), dtype=jnp.float32)

    def outer(i2, carry):
        i = i2 * 2
        carry = do_iter(i, 0, carry, True, i >= 2)
        carry = do_iter(i + 1, 1, carry, i + 3 < NCH1, i + 1 >= 2)
        return carry

    carry = lax.fori_loop(0, (NCH1 - 1) // 2, outer, (m0, s0))
    m_run, s_run = do_iter(NCH1 - 1, 0, carry, False, True)
    pltpu.make_async_copy(attb.at[1, pl.ds(0, C1)], att_out.at[pl.ds(0, C1)],
                          sem_w[1]).wait()
    pltpu.make_async_copy(attb.at[0, pl.ds(0, C1)], att_out.at[pl.ds(0, C1)],
                          sem_w[0]).wait()

    m_fin = _lane_max(m_run)
    s_fin = _lane_sum(s_run * jnp.exp(m_run - m_fin))
    statb[...] = m_fin
    pltpu.sync_copy(statb, m_out.at[pl.ds(wid * 16, 16)])
    statb[...] = s_fin
    pltpu.sync_copy(statb, s_out.at[pl.ds(wid * 16, 16)])


@functools.partial(
    pl.kernel,
    out_type=jax.ShapeDtypeStruct((NC, N, D), jnp.float32),
    mesh=_SC_MESH,
    scratch_types=[
        pltpu.VMEM((2, C), jnp.int32),
        pltpu.VMEM((2, C), jnp.int32),
        pltpu.VMEM((C,), jnp.int32),
        pltpu.VMEM((2, C, D), jnp.float32),
        pltpu.VMEM((2, C, D), jnp.float32),
        pltpu.VMEM((2, C, D), jnp.float32),
        pltpu.VMEM((2, CP), jnp.float32),
        pltpu.VMEM((64,), jnp.float32),
        pltpu.VMEM((2, C, D), jnp.float32),
        pltpu.VMEM((NW * 16,), jnp.float32),
        pltpu.VMEM((NW * 16,), jnp.float32),
        pltpu.SemaphoreType.DMA,
        pltpu.SemaphoreType.DMA,
        pltpu.SemaphoreType.DMA,
        pltpu.SemaphoreType.DMA,
        pltpu.SemaphoreType.DMA,
        pltpu.VMEM_SHARED((N, D), jnp.float32),
    ],
)
def _s2(src_hbm, dst_hbm, v_hbm, hi_hbm, hj_hbm, ea_hbm, att_hbm,
        m_hbm, s_hbm, zseg_hbm, aggr_out,
        sidx, didx, didx_sc, vrows, hirows, hjrows, attb, pbuf, msgb,
        mtab, stab, sem_i0, sem_i1, sem_g0, sem_g1, sem_sc, aggr):
    cid = lax.axis_index("c")
    sid = lax.axis_index("s")
    wid = sid * NC + cid
    base_w = wid * EPW
    sem_i = (sem_i0, sem_i1)
    sem_g = (sem_g0, sem_g1)

    pltpu.sync_copy(zseg_hbm, aggr.at[pl.ds(sid * RPT, RPT)])

    @pl.when(sid == NS - 1)
    def _zero_tail():
        pltpu.sync_copy(zseg_hbm.at[pl.ds(0, REM)],
                        aggr.at[pl.ds(NS * RPT, REM)])

    pltpu.sync_copy(m_hbm, mtab)
    pltpu.sync_copy(s_hbm, stab)

    def mred(w, m_r):
        return jnp.maximum(m_r, mtab[pl.ds(w * 16, 16)])

    m_glob = lax.fori_loop(0, NW, mred, jnp.full((16,), -1e30, jnp.float32))

    def sred(w, s_r):
        return s_r + stab[pl.ds(w * 16, 16)] * jnp.exp(mtab[pl.ds(w * 16, 16)] - m_glob)

    s_glob = lax.fori_loop(0, NW, sred, jnp.zeros((16,), jnp.float32))
    inv_s = 1.0 / s_glob

    plsc.subcore_barrier()

    def issue_idx(i, slot):
        base = base_w + i * C
        pltpu.async_copy(src_hbm.at[pl.ds(base, C)], sidx.at[slot], sem_i[slot])
        pltpu.async_copy(dst_hbm.at[pl.ds(base, C)], didx.at[slot], sem_i[slot])

    def wait_idx(slot):
        pltpu.make_async_copy(src_hbm.at[pl.ds(0, C)], sidx.at[slot],
                              sem_i[slot]).wait()
        pltpu.make_async_copy(dst_hbm.at[pl.ds(0, C)], didx.at[slot],
                              sem_i[slot]).wait()

    def issue_gathers(i, slot):
        base = base_w + i * C
        pltpu.async_copy(v_hbm.at[sidx.at[slot]], vrows.at[slot], sem_g[slot])
        pltpu.async_copy(hi_hbm.at[sidx.at[slot]], hirows.at[slot], sem_g[slot])
        pltpu.async_copy(hj_hbm.at[didx.at[slot]], hjrows.at[slot], sem_g[slot])
        pltpu.async_copy(att_hbm.at[pl.ds(base, C)],
                         attb.at[slot, pl.ds(0, C)], sem_g[slot])

    def issue_ea(i, slot):
        pltpu.async_copy(ea_hbm.at[pl.ds(base_w + i * C, C)], msgb.at[slot],
                         sem_g[slot])

    def wait_gathers(slot):
        pltpu.make_async_copy(v_hbm.at[sidx.at[slot]], vrows.at[slot],
                              sem_g[slot]).wait()
        pltpu.make_async_copy(hi_hbm.at[sidx.at[slot]], hirows.at[slot],
                              sem_g[slot]).wait()
        pltpu.make_async_copy(hj_hbm.at[didx.at[slot]], hjrows.at[slot],
                              sem_g[slot]).wait()
        pltpu.make_async_copy(att_hbm.at[pl.ds(0, C)],
                              attb.at[slot, pl.ds(0, C)], sem_g[slot]).wait()
        pltpu.make_async_copy(ea_hbm.at[pl.ds(0, C)], msgb.at[slot],
                              sem_g[slot]).wait()

    def wait_scatter():
        pltpu.make_async_copy(msgb.at[0], aggr.at[didx_sc], sem_sc).wait()

    def do_iter(i, b, p_sc, p_idx2):
        nb = 1 - b
        if not isinstance(i, int) or i + 1 < NCHUNK:
            wait_idx(nb)
            issue_gathers(i + 1, nb)
        wait_gathers(b)
        _maybe_when(p_sc, wait_scatter)
        for st in (0, 16, C - 16):
            didx_sc[st:st + 16] = didx[b, st:st + 16]
        _maybe_when(p_idx2, lambda: issue_idx(i + 2, b))
        if not isinstance(i, int) or i + 1 < NCHUNK:
            issue_ea(i + 1, nb)

        for t in range(3):
            av = attb[b, t * 16:t * 16 + 16]
            pbuf[t * 16:t * 16 + 16] = jnp.exp(av - m_glob) * inv_s

        def one_edge(j):
            p = pbuf[pl.ds(j, 16)][0]
            for r in range(8):
                sl = pl.ds(16 * r, 16)
                z = msgb[b, j, sl] + hirows[b, j, sl] + hjrows[b, j, sl]
                gate = 1.0 / (1.0 + jnp.exp(-z))
                msgb[b, j, sl] = p * vrows[b, j, sl] * gate

        def edge(jj, _):
            for u in range(4):
                one_edge(jj * 4 + u)
            return 0

        lax.fori_loop(0, C // 4, edge, 0)
        pltpu.async_copy(msgb.at[b], aggr.at[didx_sc], sem_sc, add=True)
        return 0

    issue_idx(0, 0)
    issue_idx(1, 1)
    wait_idx(0)
    issue_gathers(0, 0)
    issue_ea(0, 0)

    def outer(i2, _):
        i = i2 * 2
        do_iter(i, 0, i >= 1, True)
        do_iter(i + 1, 1, True, True)
        return 0

    lax.fori_loop(0, (NCHUNK - 2) // 2, outer, 0)
    do_iter(NCHUNK - 2, 0, True, False)
    do_iter(NCHUNK - 1, 1, True, False)
    wait_scatter()

    plsc.subcore_barrier()
    pltpu.sync_copy(aggr.at[pl.ds(sid * RPT, RPT)],
                    aggr_out.at[cid, pl.ds(sid * RPT, RPT)])

    @pl.when(sid == NS - 1)
    def _export_tail():
        pltpu.sync_copy(aggr.at[pl.ds(NS * RPT, REM)],
                        aggr_out.at[cid, pl.ds(NS * RPT, REM)])


def _ln(y, g, b):
    m = jnp.mean(y, axis=-1, keepdims=True)
    var = jnp.mean((y - m) ** 2, axis=-1, keepdims=True)
    return (y - m) / jnp.sqrt(var + 1e-5) * g + b


def _tail_body(ag_ref, root_ref, g1_ref, b1g_ref, W1_ref, b1_ref, W2_ref,
               b2_ref, g2_ref, b2g_ref, wf_ref, bf_ref, out_ref):
    a = ag_ref[0] + ag_ref[1] + root_ref[...]
    ss = _ln(a, g1_ref[...], b1g_ref[...])
    h = jnp.maximum(
        jnp.dot(ss, W1_ref[...], preferred_element_type=jnp.float32)
        + b1_ref[...], 0.0)
    h2 = jnp.dot(h, W2_ref[...], preferred_element_type=jnp.float32) + b2_ref[...]
    o = _ln(a + h2, g2_ref[...], b2g_ref[...])
    y = jnp.dot(o, wf_ref[...], preferred_element_type=jnp.float32) + bf_ref[...]
    out_ref[...] = jnp.where(y >= 0, y, 0.01 * y)


def _tail(aggr2, root, ln1_g, ln1_b, W1, b1, W2, b2, ln2_g, ln2_b, wf, bf):
    return pl.pallas_call(
        _tail_body,
        grid=(_GRID,),
        in_specs=[
            pl.BlockSpec((NC, _BLK, D), lambda i: (0, i, 0)),
            pl.BlockSpec((_BLK, D), lambda i: (i, 0)),
            pl.BlockSpec((1, D), lambda i: (0, 0)),
            pl.BlockSpec((1, D), lambda i: (0, 0)),
            pl.BlockSpec((D, 512), lambda i: (0, 0)),
            pl.BlockSpec((1, 512), lambda i: (0, 0)),
            pl.BlockSpec((512, D), lambda i: (0, 0)),
            pl.BlockSpec((1, D), lambda i: (0, 0)),
            pl.BlockSpec((1, D), lambda i: (0, 0)),
            pl.BlockSpec((1, D), lambda i: (0, 0)),
            pl.BlockSpec((D, D), lambda i: (0, 0)),
            pl.BlockSpec((1, D), lambda i: (0, 0)),
        ],
        out_specs=pl.BlockSpec((_BLK, D), lambda i: (i, 0)),
        out_shape=jax.ShapeDtypeStruct((N, D), jnp.float32),
    )(aggr2, root, ln1_g, ln1_b, W1, b1, W2, b2, ln2_g, ln2_b, wf, bf)


def kernel(x, edge_index, edge_attr, Wq, bq, Wk, bk, Wv, bv, Wr, br, Whi, Whj,
           ln1_g, ln1_b, W1, b1, W2, b2, ln2_g, ln2_b, Wl, bl, Wl2, bl2):
    w_all = jnp.concatenate([Wq, Wk, Wv, Whi, Whj, Wr], axis=1)
    zb = jnp.zeros_like(bq)
    b_all = jnp.concatenate([bq, bk, bv, zb, zb, br])[None, :]
    q, k, v, hi, hj, root, wf, bf = _proj(x, w_all, b_all, Wl, Wl2,
                                          bl[None, :], bl2[None, :])

    src = edge_index[0]
    dst = edge_index[1]
    att, m_w, s_w = _s1(src, dst, q, k)
    zseg = jnp.zeros((RPT, D), jnp.float32)
    aggr2 = _s2(src, dst, v, hi, hj, edge_attr, att, m_w, s_w, zseg)

    return _tail(aggr2, root, ln1_g[None, :], ln1_b[None, :], W1, b1[None, :],
                 W2, b2[None, :], ln2_g[None, :], ln2_b[None, :], wf, bf)


# speedup vs baseline: 1.6233x; 1.6233x over previous
import functools

import jax
import jax.numpy as jnp
from jax import lax
from jax.experimental import pallas as pl
from jax.experimental.pallas import tpu as pltpu
from jax.experimental.pallas import tpu_sc as plsc

N = 10000
E = 320000
D = 128
H1 = 1546
NC = 2
NS = 16
NW = NC * NS
EPW = E // NW
C = 40
NCHUNK = EPW // C
CP = 48
C1 = 80
NCH1 = EPW // C1
RPT = 624
REM = N - RPT * NS

_SC_MESH = plsc.VectorSubcoreMesh(core_axis_name="c", subcore_axis_name="s")


def _shuffle(v, sh):
    idx = (jnp.arange(16, dtype=jnp.int32) ^ sh)[:, None]
    return lax.gather(
        v, idx,
        dimension_numbers=lax.GatherDimensionNumbers(
            offset_dims=(), collapsed_slice_dims=(0,), start_index_map=(0,)),
        slice_sizes=(1,),
        mode=lax.GatherScatterMode.PROMISE_IN_BOUNDS)


def _lane_sum(v):
    for sh in (1, 2, 4, 8):
        v = v + _shuffle(v, sh)
    return v


def _lane_max(v):
    for sh in (1, 2, 4, 8):
        v = jnp.maximum(v, _shuffle(v, sh))
    return v


_BLK = 1000
_GRID = N // _BLK


def _proj_body(x_ref, w_ref, b_ref, q_ref, k_ref, v_ref, hi_ref, hj_ref,
               root_ref):
    y = jnp.dot(x_ref[...], w_ref[...], preferred_element_type=jnp.float32)
    y = y + b_ref[...]
    q_ref[...] = y[:, 0:128]
    k_ref[...] = y[:, 128:256]
    v_ref[...] = y[:, 256:384]
    hi_ref[...] = y[:, 384:512]
    hj_ref[...] = y[:, 512:640]
    root_ref[...] = y[:, 640:768]


def _proj(x, w_all, b_all):
    outs = [jax.ShapeDtypeStruct((N, D), jnp.float32) for _ in range(6)]
    return pl.pallas_call(
        _proj_body,
        grid=(_GRID,),
        in_specs=[
            pl.BlockSpec((_BLK, D), lambda i: (i, 0)),
            pl.BlockSpec((D, 768), lambda i: (0, 0)),
            pl.BlockSpec((1, 768), lambda i: (0, 0)),
        ],
        out_specs=[pl.BlockSpec((_BLK, D), lambda i: (i, 0))] * 6,
        out_shape=outs,
    )(x, w_all, b_all)


def _wfuse_body(wl_ref, wl2_ref, bl_ref, bl2_ref, wf_ref, bf_ref):
    wf_ref[...] = jnp.dot(wl_ref[...], wl2_ref[...],
                          preferred_element_type=jnp.float32)
    bf_ref[...] = jnp.dot(bl_ref[...], wl2_ref[...],
                          preferred_element_type=jnp.float32) + bl2_ref[...]


def _wfuse(Wl, Wl2, bl, bl2):
    return pl.pallas_call(
        _wfuse_body,
        out_shape=[jax.ShapeDtypeStruct((D, D), jnp.float32),
                   jax.ShapeDtypeStruct((1, D), jnp.float32)],
    )(Wl, Wl2, bl, bl2)


def _maybe_when(pred, fn):
    if isinstance(pred, bool):
        if pred:
            fn()
        return
    pl.when(pred)(fn)


@functools.partial(
    pl.kernel,
    out_type=[
        jax.ShapeDtypeStruct((E,), jnp.float32),
        jax.ShapeDtypeStruct((NW * 16,), jnp.float32),
        jax.ShapeDtypeStruct((NW * 16,), jnp.float32),
    ],
    mesh=_SC_MESH,
    scratch_types=[
        pltpu.VMEM((2, C1), jnp.int32),
        pltpu.VMEM((2, C1), jnp.int32),
        pltpu.VMEM((2, C1, D), jnp.float32),
        pltpu.VMEM((2, C1, D), jnp.float32),
        pltpu.VMEM((2, C1 + 16), jnp.float32),
        pltpu.VMEM((16,), jnp.float32),
        pltpu.SemaphoreType.DMA,
        pltpu.SemaphoreType.DMA,
        pltpu.SemaphoreType.DMA,
        pltpu.SemaphoreType.DMA,
        pltpu.SemaphoreType.DMA,
        pltpu.SemaphoreType.DMA,
    ],
)
def _s1(src_hbm, dst_hbm, q_hbm, k_hbm, att_out, m_out, s_out,
        sidx, didx, qrows, krows, attb, statb,
        sem_i0, sem_i1, sem_g0, sem_g1, sem_w0, sem_w1):
    cid = lax.axis_index("c")
    sid = lax.axis_index("s")
    wid = sid * NC + cid
    base_w = wid * EPW
    sem_i = (sem_i0, sem_i1)
    sem_g = (sem_g0, sem_g1)
    sem_w = (sem_w0, sem_w1)
    lane = jnp.arange(16, dtype=jnp.int32)

    def issue_idx(i, slot):
        base = base_w + i * C1
        pltpu.async_copy(src_hbm.at[pl.ds(base, C1)], sidx.at[slot],
                         sem_i[slot])
        pltpu.async_copy(dst_hbm.at[pl.ds(base, C1)], didx.at[slot],
                         sem_i[slot])

    def wait_idx(slot):
        pltpu.make_async_copy(src_hbm.at[pl.ds(0, C1)], sidx.at[slot],
                              sem_i[slot]).wait()
        pltpu.make_async_copy(dst_hbm.at[pl.ds(0, C1)], didx.at[slot],
                              sem_i[slot]).wait()

    def issue_gathers(slot):
        pltpu.async_copy(q_hbm.at[didx.at[slot]], qrows.at[slot], sem_g[slot])
        pltpu.async_copy(k_hbm.at[sidx.at[slot]], krows.at[slot], sem_g[slot])

    def wait_gathers(slot):
        pltpu.make_async_copy(q_hbm.at[didx.at[slot]], qrows.at[slot],
                              sem_g[slot]).wait()
        pltpu.make_async_copy(k_hbm.at[sidx.at[slot]], krows.at[slot],
                              sem_g[slot]).wait()

    def do_iter(i, b, carry, p_idx2, p_attw):
        m_r, s_r = carry
        nb = 1 - b
        if not isinstance(i, int) or i + 1 < NCH1:
            wait_idx(nb)
            issue_gathers(nb)
        _maybe_when(p_attw, lambda: pltpu.make_async_copy(
            attb.at[b, pl.ds(0, C1)], att_out.at[pl.ds(0, C1)],
            sem_w[b]).wait())
        wait_gathers(b)
        _maybe_when(p_idx2, lambda: issue_idx(i + 2, b))

        def one_edge(j):
            acc = qrows[b, j, 0:16] * krows[b, j, 0:16]
            for r in range(1, 8):
                acc = acc + (qrows[b, j, 16 * r:16 * r + 16]
                             * krows[b, j, 16 * r:16 * r + 16])
            attb[b, pl.ds(j, 16)] = _lane_sum(acc)

        def edge(jj, _):
            one_edge(jj * 2)
            one_edge(jj * 2 + 1)
            return 0

        lax.fori_loop(0, C1 // 2, edge, 0)
        for t in range(C1 // 16):
            vec = attb[b, t * 16:t * 16 + 16]
            m_n = jnp.maximum(m_r, vec)
            s_r = s_r * jnp.exp(m_r - m_n) + jnp.exp(vec - m_n)
            m_r = m_n
        pltpu.async_copy(attb.at[b, pl.ds(0, C1)],
                         att_out.at[pl.ds(base_w + i * C1, C1)], sem_w[b])
        return (m_r, s_r)

    issue_idx(0, 0)
    issue_idx(1, 1)
    wait_idx(0)
    issue_gathers(0)

    m0 = jnp.full((16,), -1e30, dtype=jnp.float32)
    s0 = jnp.zeros((16,), dtype=jnp.float32)

    def outer(i2, carry):
        i = i2 * 2
        carry = do_iter(i, 0, carry, True, i >= 2)
        carry = do_iter(i + 1, 1, carry, i + 3 < NCH1, i + 1 >= 2)
        return carry

    carry = lax.fori_loop(0, (NCH1 - 1) // 2, outer, (m0, s0))
    m_run, s_run = do_iter(NCH1 - 1, 0, carry, False, True)
    pltpu.make_async_copy(attb.at[1, pl.ds(0, C1)], att_out.at[pl.ds(0, C1)],
                          sem_w[1]).wait()
    pltpu.make_async_copy(attb.at[0, pl.ds(0, C1)], att_out.at[pl.ds(0, C1)],
                          sem_w[0]).wait()

    m_fin = _lane_max(m_run)
    s_fin = _lane_sum(s_run * jnp.exp(m_run - m_fin))
    statb[...] = m_fin
    pltpu.sync_copy(statb, m_out.at[pl.ds(wid * 16, 16)])
    statb[...] = s_fin
    pltpu.sync_copy(statb, s_out.at[pl.ds(wid * 16, 16)])


@functools.partial(
    pl.kernel,
    out_type=jax.ShapeDtypeStruct((NC, N, D), jnp.float32),
    mesh=_SC_MESH,
    scratch_types=[
        pltpu.VMEM((2, C), jnp.int32),
        pltpu.VMEM((2, C), jnp.int32),
        pltpu.VMEM((C,), jnp.int32),
        pltpu.VMEM((2, C, D), jnp.float32),
        pltpu.VMEM((2, C, D), jnp.float32),
        pltpu.VMEM((2, C, D), jnp.float32),
        pltpu.VMEM((2, CP), jnp.float32),
        pltpu.VMEM((64,), jnp.float32),
        pltpu.VMEM((2, C, D), jnp.float32),
        pltpu.VMEM((NW * 16,), jnp.float32),
        pltpu.VMEM((NW * 16,), jnp.float32),
        pltpu.SemaphoreType.DMA,
        pltpu.SemaphoreType.DMA,
        pltpu.SemaphoreType.DMA,
        pltpu.SemaphoreType.DMA,
        pltpu.SemaphoreType.DMA,
        pltpu.VMEM_SHARED((N, D), jnp.float32),
    ],
)
def _s2(src_hbm, dst_hbm, v_hbm, hi_hbm, hj_hbm, ea_hbm, att_hbm,
        m_hbm, s_hbm, zseg_hbm, aggr_out,
        sidx, didx, didx_sc, vrows, hirows, hjrows, attb, pbuf, msgb,
        mtab, stab, sem_i0, sem_i1, sem_g0, sem_g1, sem_sc, aggr):
    cid = lax.axis_index("c")
    sid = lax.axis_index("s")
    wid = sid * NC + cid
    base_w = wid * EPW
    sem_i = (sem_i0, sem_i1)
    sem_g = (sem_g0, sem_g1)

    pltpu.sync_copy(zseg_hbm, aggr.at[pl.ds(sid * RPT, RPT)])

    @pl.when(sid == NS - 1)
    def _zero_tail():
        pltpu.sync_copy(zseg_hbm.at[pl.ds(0, REM)],
                        aggr.at[pl.ds(NS * RPT, REM)])

    pltpu.sync_copy(m_hbm, mtab)
    pltpu.sync_copy(s_hbm, stab)

    def mred(w, m_r):
        return jnp.maximum(m_r, mtab[pl.ds(w * 16, 16)])

    m_glob = lax.fori_loop(0, NW, mred, jnp.full((16,), -1e30, jnp.float32))

    def sred(w, s_r):
        return s_r + stab[pl.ds(w * 16, 16)] * jnp.exp(mtab[pl.ds(w * 16, 16)] - m_glob)

    s_glob = lax.fori_loop(0, NW, sred, jnp.zeros((16,), jnp.float32))
    inv_s = 1.0 / s_glob

    plsc.subcore_barrier()

    def issue_idx(i, slot):
        base = base_w + i * C
        pltpu.async_copy(src_hbm.at[pl.ds(base, C)], sidx.at[slot], sem_i[slot])
        pltpu.async_copy(dst_hbm.at[pl.ds(base, C)], didx.at[slot], sem_i[slot])

    def wait_idx(slot):
        pltpu.make_async_copy(src_hbm.at[pl.ds(0, C)], sidx.at[slot],
                              sem_i[slot]).wait()
        pltpu.make_async_copy(dst_hbm.at[pl.ds(0, C)], didx.at[slot],
                              sem_i[slot]).wait()

    def issue_gathers(i, slot):
        base = base_w + i * C
        pltpu.async_copy(v_hbm.at[sidx.at[slot]], vrows.at[slot], sem_g[slot])
        pltpu.async_copy(hi_hbm.at[sidx.at[slot]], hirows.at[slot], sem_g[slot])
        pltpu.async_copy(hj_hbm.at[didx.at[slot]], hjrows.at[slot], sem_g[slot])
        pltpu.async_copy(att_hbm.at[pl.ds(base, C)],
                         attb.at[slot, pl.ds(0, C)], sem_g[slot])

    def issue_ea(i, slot):
        pltpu.async_copy(ea_hbm.at[pl.ds(base_w + i * C, C)], msgb.at[slot],
                         sem_g[slot])

    def wait_gathers(slot):
        pltpu.make_async_copy(v_hbm.at[sidx.at[slot]], vrows.at[slot],
                              sem_g[slot]).wait()
        pltpu.make_async_copy(hi_hbm.at[sidx.at[slot]], hirows.at[slot],
                              sem_g[slot]).wait()
        pltpu.make_async_copy(hj_hbm.at[didx.at[slot]], hjrows.at[slot],
                              sem_g[slot]).wait()
        pltpu.make_async_copy(att_hbm.at[pl.ds(0, C)],
                              attb.at[slot, pl.ds(0, C)], sem_g[slot]).wait()
        pltpu.make_async_copy(ea_hbm.at[pl.ds(0, C)], msgb.at[slot],
                              sem_g[slot]).wait()

    def wait_scatter():
        pltpu.make_async_copy(msgb.at[0], aggr.at[didx_sc], sem_sc).wait()

    def do_iter(i, b, p_sc, p_idx2):
        nb = 1 - b
        if not isinstance(i, int) or i + 1 < NCHUNK:
            wait_idx(nb)
            issue_gathers(i + 1, nb)
        wait_gathers(b)
        _maybe_when(p_sc, wait_scatter)
        for st in (0, 16, C - 16):
            didx_sc[st:st + 16] = didx[b, st:st + 16]
        _maybe_when(p_idx2, lambda: issue_idx(i + 2, b))
        if not isinstance(i, int) or i + 1 < NCHUNK:
            issue_ea(i + 1, nb)

        for t in range(3):
            av = attb[b, t * 16:t * 16 + 16]
            pbuf[t * 16:t * 16 + 16] = jnp.exp(av - m_glob) * inv_s

        def one_edge(j):
            p = pbuf[pl.ds(j, 16)][0]
            for r in range(8):
                sl = pl.ds(16 * r, 16)
                z = msgb[b, j, sl] + hirows[b, j, sl] + hjrows[b, j, sl]
                gate = 1.0 / (1.0 + jnp.exp(-z))
                msgb[b, j, sl] = p * vrows[b, j, sl] * gate

        def edge(jj, _):
            one_edge(jj * 2)
            one_edge(jj * 2 + 1)
            return 0

        lax.fori_loop(0, C // 2, edge, 0)
        pltpu.async_copy(msgb.at[b], aggr.at[didx_sc], sem_sc, add=True)
        return 0

    issue_idx(0, 0)
    issue_idx(1, 1)
    wait_idx(0)
    issue_gathers(0, 0)
    issue_ea(0, 0)

    def outer(i2, _):
        i = i2 * 2
        do_iter(i, 0, i >= 1, True)
        do_iter(i + 1, 1, True, True)
        return 0

    lax.fori_loop(0, (NCHUNK - 2) // 2, outer, 0)
    do_iter(NCHUNK - 2, 0, True, False)
    do_iter(NCHUNK - 1, 1, True, False)
    wait_scatter()

    plsc.subcore_barrier()
    pltpu.sync_copy(aggr.at[pl.ds(sid * RPT, RPT)],
                    aggr_out.at[cid, pl.ds(sid * RPT, RPT)])

    @pl.when(sid == NS - 1)
    def _export_tail():
        pltpu.sync_copy(aggr.at[pl.ds(NS * RPT, REM)],
                        aggr_out.at[cid, pl.ds(NS * RPT, REM)])


def _ln(y, g, b):
    m = jnp.mean(y, axis=-1, keepdims=True)
    var = jnp.mean((y - m) ** 2, axis=-1, keepdims=True)
    return (y - m) / jnp.sqrt(var + 1e-5) * g + b


def _tail_body(ag_ref, root_ref, g1_ref, b1g_ref, W1_ref, b1_ref, W2_ref,
               b2_ref, g2_ref, b2g_ref, wf_ref, bf_ref, out_ref):
    a = ag_ref[0] + ag_ref[1] + root_ref[...]
    ss = _ln(a, g1_ref[...], b1g_ref[...])
    h = jnp.maximum(
        jnp.dot(ss, W1_ref[...], preferred_element_type=jnp.float32)
        + b1_ref[...], 0.0)
    h2 = jnp.dot(h, W2_ref[...], preferred_element_type=jnp.float32) + b2_ref[...]
    o = _ln(a + h2, g2_ref[...], b2g_ref[...])
    y = jnp.dot(o, wf_ref[...], preferred_element_type=jnp.float32) + bf_ref[...]
    out_ref[...] = jnp.where(y >= 0, y, 0.01 * y)


def _tail(aggr2, root, ln1_g, ln1_b, W1, b1, W2, b2, ln2_g, ln2_b, wf, bf):
    return pl.pallas_call(
        _tail_body,
        grid=(_GRID,),
        in_specs=[
            pl.BlockSpec((NC, _BLK, D), lambda i: (0, i, 0)),
            pl.BlockSpec((_BLK, D), lambda i: (i, 0)),
            pl.BlockSpec((1, D), lambda i: (0, 0)),
            pl.BlockSpec((1, D), lambda i: (0, 0)),
            pl.BlockSpec((D, 512), lambda i: (0, 0)),
            pl.BlockSpec((1, 512), lambda i: (0, 0)),
            pl.BlockSpec((512, D), lambda i: (0, 0)),
            pl.BlockSpec((1, D), lambda i: (0, 0)),
            pl.BlockSpec((1, D), lambda i: (0, 0)),
            pl.BlockSpec((1, D), lambda i: (0, 0)),
            pl.BlockSpec((D, D), lambda i: (0, 0)),
            pl.BlockSpec((1, D), lambda i: (0, 0)),
        ],
        out_specs=pl.BlockSpec((_BLK, D), lambda i: (i, 0)),
        out_shape=jax.ShapeDtypeStruct((N, D), jnp.float32),
    )(aggr2, root, ln1_g, ln1_b, W1, b1, W2, b2, ln2_g, ln2_b, wf, bf)


def kernel(x, edge_index, edge_attr, Wq, bq, Wk, bk, Wv, bv, Wr, br, Whi, Whj,
           ln1_g, ln1_b, W1, b1, W2, b2, ln2_g, ln2_b, Wl, bl, Wl2, bl2):
    w_all = jnp.concatenate([Wq, Wk, Wv, Whi, Whj, Wr], axis=1)
    zb = jnp.zeros_like(bq)
    b_all = jnp.concatenate([bq, bk, bv, zb, zb, br])[None, :]
    q, k, v, hi, hj, root = _proj(x, w_all, b_all)
    wf, bf = _wfuse(Wl, Wl2, bl[None, :], bl2[None, :])

    src = edge_index[0]
    dst = edge_index[1]
    att, m_w, s_w = _s1(src, dst, q, k)
    zseg = jnp.zeros((RPT, D), jnp.float32)
    aggr2 = _s2(src, dst, v, hi, hj, edge_attr, att, m_w, s_w, zseg)

    return _tail(aggr2, root, ln1_g[None, :], ln1_b[None, :], W1, b1[None, :],
                 W2, b2[None, :], ln2_g[None, :], ln2_b[None, :], wf, bf)


# speedup vs baseline: 1.6257x; 1.0015x over previous
import functools

import jax
import jax.numpy as jnp
from jax import lax
from jax.experimental import pallas as pl
from jax.experimental.pallas import tpu as pltpu
from jax.experimental.pallas import tpu_sc as plsc

N = 10000
E = 320000
D = 128
H1 = 1546
NC = 2
NS = 16
NW = NC * NS
EPW = E // NW
C = 40
NCHUNK = EPW // C
CP = 48
C1 = 80
NCH1 = EPW // C1
RPT = 624
REM = N - RPT * NS

_SC_MESH = plsc.VectorSubcoreMesh(core_axis_name="c", subcore_axis_name="s")


def _shuffle(v, sh):
    idx = (jnp.arange(16, dtype=jnp.int32) ^ sh)[:, None]
    return lax.gather(
        v, idx,
        dimension_numbers=lax.GatherDimensionNumbers(
            offset_dims=(), collapsed_slice_dims=(0,), start_index_map=(0,)),
        slice_sizes=(1,),
        mode=lax.GatherScatterMode.PROMISE_IN_BOUNDS)


def _lane_sum(v):
    for sh in (1, 2, 4, 8):
        v = v + _shuffle(v, sh)
    return v


def _lane_max(v):
    for sh in (1, 2, 4, 8):
        v = jnp.maximum(v, _shuffle(v, sh))
    return v


_BLK = 1000
_GRID = N // _BLK


def _proj_body(x_ref, w_ref, b_ref, q_ref, k_ref, v_ref, hi_ref, hj_ref,
               root_ref):
    y = jnp.dot(x_ref[...], w_ref[...], preferred_element_type=jnp.float32)
    y = y + b_ref[...]
    q_ref[...] = y[:, 0:128]
    k_ref[...] = y[:, 128:256]
    v_ref[...] = y[:, 256:384]
    hi_ref[...] = y[:, 384:512]
    hj_ref[...] = y[:, 512:640]
    root_ref[...] = y[:, 640:768]


def _proj(x, w_all, b_all):
    outs = [jax.ShapeDtypeStruct((N, D), jnp.float32) for _ in range(6)]
    return pl.pallas_call(
        _proj_body,
        grid=(_GRID,),
        in_specs=[
            pl.BlockSpec((_BLK, D), lambda i: (i, 0)),
            pl.BlockSpec((D, 768), lambda i: (0, 0)),
            pl.BlockSpec((1, 768), lambda i: (0, 0)),
        ],
        out_specs=[pl.BlockSpec((_BLK, D), lambda i: (i, 0))] * 6,
        out_shape=outs,
    )(x, w_all, b_all)


def _wfuse_body(wl_ref, wl2_ref, bl_ref, bl2_ref, wf_ref, bf_ref):
    wf_ref[...] = jnp.dot(wl_ref[...], wl2_ref[...],
                          preferred_element_type=jnp.float32)
    bf_ref[...] = jnp.dot(bl_ref[...], wl2_ref[...],
                          preferred_element_type=jnp.float32) + bl2_ref[...]


def _wfuse(Wl, Wl2, bl, bl2):
    return pl.pallas_call(
        _wfuse_body,
        out_shape=[jax.ShapeDtypeStruct((D, D), jnp.float32),
                   jax.ShapeDtypeStruct((1, D), jnp.float32)],
    )(Wl, Wl2, bl, bl2)


def _maybe_when(pred, fn):
    if isinstance(pred, bool):
        if pred:
            fn()
        return
    pl.when(pred)(fn)


@functools.partial(
    pl.kernel,
    out_type=[
        jax.ShapeDtypeStruct((E,), jnp.float32),
        jax.ShapeDtypeStruct((NW * 16,), jnp.float32),
        jax.ShapeDtypeStruct((NW * 16,), jnp.float32),
    ],
    mesh=_SC_MESH,
    scratch_types=[
        pltpu.VMEM((2, C1), jnp.int32),
        pltpu.VMEM((2, C1), jnp.int32),
        pltpu.VMEM((2, C1, D), jnp.float32),
        pltpu.VMEM((2, C1, D), jnp.float32),
        pltpu.VMEM((2, C1 + 16), jnp.float32),
        pltpu.VMEM((16,), jnp.float32),
        pltpu.SemaphoreType.DMA,
        pltpu.SemaphoreType.DMA,
        pltpu.SemaphoreType.DMA,
        pltpu.SemaphoreType.DMA,
        pltpu.SemaphoreType.DMA,
        pltpu.SemaphoreType.DMA,
    ],
)
def _s1(src_hbm, dst_hbm, q_hbm, k_hbm, att_out, m_out, s_out,
        sidx, didx, qrows, krows, attb, statb,
        sem_i0, sem_i1, sem_g0, sem_g1, sem_w0, sem_w1):
    cid = lax.axis_index("c")
    sid = lax.axis_index("s")
    wid = sid * NC + cid
    base_w = wid * EPW
    sem_i = (sem_i0, sem_i1)
    sem_g = (sem_g0, sem_g1)
    sem_w = (sem_w0, sem_w1)

    def issue_idx(i, slot):
        base = base_w + i * C1
        pltpu.async_copy(src_hbm.at[pl.ds(base, C1)], sidx.at[slot],
                         sem_i[slot])
        pltpu.async_copy(dst_hbm.at[pl.ds(base, C1)], didx.at[slot],
                         sem_i[slot])

    def wait_idx(slot):
        pltpu.make_async_copy(src_hbm.at[pl.ds(0, C1)], sidx.at[slot],
                              sem_i[slot]).wait()
        pltpu.make_async_copy(dst_hbm.at[pl.ds(0, C1)], didx.at[slot],
                              sem_i[slot]).wait()

    def issue_gathers(slot):
        pltpu.async_copy(q_hbm.at[didx.at[slot]], qrows.at[slot], sem_g[slot])
        pltpu.async_copy(k_hbm.at[sidx.at[slot]], krows.at[slot], sem_g[slot])

    def wait_gathers(slot):
        pltpu.make_async_copy(q_hbm.at[didx.at[slot]], qrows.at[slot],
                              sem_g[slot]).wait()
        pltpu.make_async_copy(k_hbm.at[sidx.at[slot]], krows.at[slot],
                              sem_g[slot]).wait()

    def do_iter(i, b, carry, p_idx2, p_attw):
        m_r, s_r = carry
        nb = 1 - b
        if not isinstance(i, int) or i + 1 < NCH1:
            wait_idx(nb)
            issue_gathers(nb)
        _maybe_when(p_attw, lambda: pltpu.make_async_copy(
            attb.at[b, pl.ds(0, C1)], att_out.at[pl.ds(0, C1)],
            sem_w[b]).wait())
        wait_gathers(b)
        _maybe_when(p_idx2, lambda: issue_idx(i + 2, b))

        def one_edge(j):
            acc = qrows[b, j, 0:16] * krows[b, j, 0:16]
            for r in range(1, 8):
                acc = acc + (qrows[b, j, 16 * r:16 * r + 16]
                             * krows[b, j, 16 * r:16 * r + 16])
            attb[b, pl.ds(j, 16)] = _lane_sum(acc)

        def edge(jj, _):
            one_edge(jj * 2)
            one_edge(jj * 2 + 1)
            return 0

        lax.fori_loop(0, C1 // 2, edge, 0)
        for t in range(C1 // 16):
            vec = attb[b, t * 16:t * 16 + 16]
            m_n = jnp.maximum(m_r, vec)
            s_r = s_r * jnp.exp(m_r - m_n) + jnp.exp(vec - m_n)
            m_r = m_n
        pltpu.async_copy(attb.at[b, pl.ds(0, C1)],
                         att_out.at[pl.ds(base_w + i * C1, C1)], sem_w[b])
        return (m_r, s_r)

    issue_idx(0, 0)
    issue_idx(1, 1)
    wait_idx(0)
    issue_gathers(0)

    m0 = jnp.full((16,), -1e30, dtype=jnp.float32)
    s0 = jnp.zeros((16,), dtype=jnp.float32)

    def outer(i2, carry):
        i = i2 * 2
        carry = do_iter(i, 0, carry, True, i >= 2)
        carry = do_iter(i + 1, 1, carry, i + 3 < NCH1, i + 1 >= 2)
        return carry

    carry = lax.fori_loop(0, (NCH1 - 1) // 2, outer, (m0, s0))
    m_run, s_run = do_iter(NCH1 - 1, 0, carry, False, True)
    pltpu.make_async_copy(attb.at[1, pl.ds(0, C1)], att_out.at[pl.ds(0, C1)],
                          sem_w[1]).wait()
    pltpu.make_async_copy(attb.at[0, pl.ds(0, C1)], att_out.at[pl.ds(0, C1)],
                          sem_w[0]).wait()

    m_fin = _lane_max(m_run)
    s_fin = _lane_sum(s_run * jnp.exp(m_run - m_fin))
    statb[...] = m_fin
    pltpu.sync_copy(statb, m_out.at[pl.ds(wid * 16, 16)])
    statb[...] = s_fin
    pltpu.sync_copy(statb, s_out.at[pl.ds(wid * 16, 16)])


@functools.partial(
    pl.kernel,
    out_type=jax.ShapeDtypeStruct((NC, N, D), jnp.float32),
    mesh=_SC_MESH,
    scratch_types=[
        pltpu.VMEM((2, C), jnp.int32),
        pltpu.VMEM((2, C), jnp.int32),
        pltpu.VMEM((C,), jnp.int32),
        pltpu.VMEM((2, C, D), jnp.float32),
        pltpu.VMEM((2, C, D), jnp.float32),
        pltpu.VMEM((2, C, D), jnp.float32),
        pltpu.VMEM((2, CP), jnp.float32),
        pltpu.VMEM((64,), jnp.float32),
        pltpu.VMEM((2, C, D), jnp.float32),
        pltpu.VMEM((NW * 16,), jnp.float32),
        pltpu.VMEM((NW * 16,), jnp.float32),
        pltpu.SemaphoreType.DMA,
        pltpu.SemaphoreType.DMA,
        pltpu.SemaphoreType.DMA,
        pltpu.SemaphoreType.DMA,
        pltpu.SemaphoreType.DMA,
        pltpu.VMEM_SHARED((N, D), jnp.float32),
    ],
)
def _s2(src_hbm, dst_hbm, v_hbm, hi_hbm, hj_hbm, ea_hbm, att_hbm,
        m_hbm, s_hbm, zseg_hbm, aggr_out,
        sidx, didx, didx_sc, vrows, hirows, hjrows, attb, pbuf, msgb,
        mtab, stab, sem_i0, sem_i1, sem_g0, sem_g1, sem_sc, aggr):
    cid = lax.axis_index("c")
    sid = lax.axis_index("s")
    wid = sid * NC + cid
    base_w = wid * EPW
    sem_i = (sem_i0, sem_i1)
    sem_g = (sem_g0, sem_g1)

    pltpu.sync_copy(zseg_hbm, aggr.at[pl.ds(sid * RPT, RPT)])

    @pl.when(sid == NS - 1)
    def _zero_tail():
        pltpu.sync_copy(zseg_hbm.at[pl.ds(0, REM)],
                        aggr.at[pl.ds(NS * RPT, REM)])

    pltpu.sync_copy(m_hbm, mtab)
    pltpu.sync_copy(s_hbm, stab)

    def mred(w, m_r):
        return jnp.maximum(m_r, mtab[pl.ds(w * 16, 16)])

    m_glob = lax.fori_loop(0, NW, mred, jnp.full((16,), -1e30, jnp.float32))

    def sred(w, s_r):
        return s_r + stab[pl.ds(w * 16, 16)] * jnp.exp(mtab[pl.ds(w * 16, 16)] - m_glob)

    s_glob = lax.fori_loop(0, NW, sred, jnp.zeros((16,), jnp.float32))
    inv_s = 1.0 / s_glob

    plsc.subcore_barrier()

    def issue_idx(i, slot):
        base = base_w + i * C
        pltpu.async_copy(src_hbm.at[pl.ds(base, C)], sidx.at[slot], sem_i[slot])
        pltpu.async_copy(dst_hbm.at[pl.ds(base, C)], didx.at[slot], sem_i[slot])

    def wait_idx(slot):
        pltpu.make_async_copy(src_hbm.at[pl.ds(0, C)], sidx.at[slot],
                              sem_i[slot]).wait()
        pltpu.make_async_copy(dst_hbm.at[pl.ds(0, C)], didx.at[slot],
                              sem_i[slot]).wait()

    def issue_gathers(i, slot):
        base = base_w + i * C
        pltpu.async_copy(v_hbm.at[sidx.at[slot]], vrows.at[slot], sem_g[slot])
        pltpu.async_copy(hi_hbm.at[sidx.at[slot]], hirows.at[slot], sem_g[slot])
        pltpu.async_copy(hj_hbm.at[didx.at[slot]], hjrows.at[slot], sem_g[slot])
        pltpu.async_copy(att_hbm.at[pl.ds(base, C)],
                         attb.at[slot, pl.ds(0, C)], sem_g[slot])

    def issue_ea(i, slot):
        pltpu.async_copy(ea_hbm.at[pl.ds(base_w + i * C, C)], msgb.at[slot],
                         sem_g[slot])

    def wait_gathers(slot):
        pltpu.make_async_copy(v_hbm.at[sidx.at[slot]], vrows.at[slot],
                              sem_g[slot]).wait()
        pltpu.make_async_copy(hi_hbm.at[sidx.at[slot]], hirows.at[slot],
                              sem_g[slot]).wait()
        pltpu.make_async_copy(hj_hbm.at[didx.at[slot]], hjrows.at[slot],
                              sem_g[slot]).wait()
        pltpu.make_async_copy(att_hbm.at[pl.ds(0, C)],
                              attb.at[slot, pl.ds(0, C)], sem_g[slot]).wait()
        pltpu.make_async_copy(ea_hbm.at[pl.ds(0, C)], msgb.at[slot],
                              sem_g[slot]).wait()

    def wait_scatter():
        pltpu.make_async_copy(msgb.at[0], aggr.at[didx_sc], sem_sc).wait()

    def do_iter(i, b, p_sc, p_idx2):
        nb = 1 - b
        if not isinstance(i, int) or i + 1 < NCHUNK:
            wait_idx(nb)
            issue_gathers(i + 1, nb)
        wait_gathers(b)
        _maybe_when(p_sc, wait_scatter)
        for st in (0, 16, C - 16):
            didx_sc[st:st + 16] = didx[b, st:st + 16]
        _maybe_when(p_idx2, lambda: issue_idx(i + 2, b))
        if not isinstance(i, int) or i + 1 < NCHUNK:
            issue_ea(i + 1, nb)

        for t in range(3):
            av = attb[b, t * 16:t * 16 + 16]
            pbuf[t * 16:t * 16 + 16] = jnp.exp(av - m_glob) * inv_s

        def one_edge(j):
            p = pbuf[pl.ds(j, 16)][0]
            for r in range(8):
                sl = pl.ds(16 * r, 16)
                z = msgb[b, j, sl] + hirows[b, j, sl] + hjrows[b, j, sl]
                gate = 1.0 / (1.0 + jnp.exp(-z))
                msgb[b, j, sl] = p * vrows[b, j, sl] * gate

        def edge(jj, _):
            one_edge(jj * 2)
            one_edge(jj * 2 + 1)
            return 0

        lax.fori_loop(0, C // 2, edge, 0)
        pltpu.async_copy(msgb.at[b], aggr.at[didx_sc], sem_sc, add=True)
        return 0

    issue_idx(0, 0)
    issue_idx(1, 1)
    wait_idx(0)
    issue_gathers(0, 0)
    issue_ea(0, 0)

    def outer(i2, _):
        i = i2 * 2
        do_iter(i, 0, i >= 1, True)
        do_iter(i + 1, 1, True, True)
        return 0

    lax.fori_loop(0, (NCHUNK - 2) // 2, outer, 0)
    do_iter(NCHUNK - 2, 0, True, False)
    do_iter(NCHUNK - 1, 1, True, False)
    wait_scatter()

    plsc.subcore_barrier()
    pltpu.sync_copy(aggr.at[pl.ds(sid * RPT, RPT)],
                    aggr_out.at[cid, pl.ds(sid * RPT, RPT)])

    @pl.when(sid == NS - 1)
    def _export_tail():
        pltpu.sync_copy(aggr.at[pl.ds(NS * RPT, REM)],
                        aggr_out.at[cid, pl.ds(NS * RPT, REM)])


def _ln(y, g, b):
    m = jnp.mean(y, axis=-1, keepdims=True)
    var = jnp.mean((y - m) ** 2, axis=-1, keepdims=True)
    return (y - m) / jnp.sqrt(var + 1e-5) * g + b


def _tail_body(ag_ref, root_ref, g1_ref, b1g_ref, W1_ref, b1_ref, W2_ref,
               b2_ref, g2_ref, b2g_ref, wf_ref, bf_ref, out_ref):
    a = ag_ref[0] + ag_ref[1] + root_ref[...]
    ss = _ln(a, g1_ref[...], b1g_ref[...])
    h = jnp.maximum(
        jnp.dot(ss, W1_ref[...], preferred_element_type=jnp.float32)
        + b1_ref[...], 0.0)
    h2 = jnp.dot(h, W2_ref[...], preferred_element_type=jnp.float32) + b2_ref[...]
    o = _ln(a + h2, g2_ref[...], b2g_ref[...])
    y = jnp.dot(o, wf_ref[...], preferred_element_type=jnp.float32) + bf_ref[...]
    out_ref[...] = jnp.where(y >= 0, y, 0.01 * y)


def _tail(aggr2, root, ln1_g, ln1_b, W1, b1, W2, b2, ln2_g, ln2_b, wf, bf):
    return pl.pallas_call(
        _tail_body,
        grid=(_GRID,),
        in_specs=[
            pl.BlockSpec((NC, _BLK, D), lambda i: (0, i, 0)),
            pl.BlockSpec((_BLK, D), lambda i: (i, 0)),
            pl.BlockSpec((1, D), lambda i: (0, 0)),
            pl.BlockSpec((1, D), lambda i: (0, 0)),
            pl.BlockSpec((D, 512), lambda i: (0, 0)),
            pl.BlockSpec((1, 512), lambda i: (0, 0)),
            pl.BlockSpec((512, D), lambda i: (0, 0)),
            pl.BlockSpec((1, D), lambda i: (0, 0)),
            pl.BlockSpec((1, D), lambda i: (0, 0)),
            pl.BlockSpec((1, D), lambda i: (0, 0)),
            pl.BlockSpec((D, D), lambda i: (0, 0)),
            pl.BlockSpec((1, D), lambda i: (0, 0)),
        ],
        out_specs=pl.BlockSpec((_BLK, D), lambda i: (i, 0)),
        out_shape=jax.ShapeDtypeStruct((N, D), jnp.float32),
    )(aggr2, root, ln1_g, ln1_b, W1, b1, W2, b2, ln2_g, ln2_b, wf, bf)


def kernel(x, edge_index, edge_attr, Wq, bq, Wk, bk, Wv, bv, Wr, br, Whi, Whj,
           ln1_g, ln1_b, W1, b1, W2, b2, ln2_g, ln2_b, Wl, bl, Wl2, bl2):
    w_all = jnp.concatenate([Wq, Wk, Wv, Whi, Whj, Wr], axis=1)
    zb = jnp.zeros_like(bq)
    b_all = jnp.concatenate([bq, bk, bv, zb, zb, br])[None, :]
    q, k, v, hi, hj, root = _proj(x, w_all, b_all)
    wf, bf = _wfuse(Wl, Wl2, bl[None, :], bl2[None, :])

    src = edge_index[0]
    dst = edge_index[1]
    att, m_w, s_w = _s1(src, dst, q, k)
    zseg = jnp.zeros((RPT, D), jnp.float32)
    aggr2 = _s2(src, dst, v, hi, hj, edge_attr, att, m_w, s_w, zseg)

    return _tail(aggr2, root, ln1_g[None, :], ln1_b[None, :], W1, b1[None, :],
                 W2, b2[None, :], ln2_g[None, :], ln2_b[None, :], wf, bf)
